# baseline (speedup 1.0000x reference)
import functools

import jax
import jax.numpy as jnp
from jax import lax
from jax.experimental import pallas as pl
from jax.experimental.pallas import tpu as pltpu
from jax.experimental.pallas import tpu_sc as plsc

D_MODEL = 1024
NUM_WORKERS = 32
CHUNK = 8
NBUF = 8
LEAD = 2


def _make_gather(batch):
    rows_per_worker = batch // NUM_WORKERS
    num_chunks = rows_per_worker // CHUNK
    num_groups = num_chunks // NBUF
    mesh = plsc.VectorSubcoreMesh(core_axis_name="c", subcore_axis_name="s")

    @functools.partial(
        pl.kernel,
        mesh=mesh,
        out_type=jax.ShapeDtypeStruct((batch, D_MODEL), jnp.float32),
        scratch_types=[
            pltpu.VMEM((rows_per_worker,), jnp.int32),
        ]
        + [pltpu.VMEM((CHUNK, D_MODEL), jnp.float32) for _ in range(NBUF)]
        + [pltpu.SemaphoreType.DMA for _ in range(2 * NBUF)],
    )
    def gather_kernel(table_hbm, idx_hbm, out_hbm, idx_v, *rest):
        bufs = rest[:NBUF]
        gsems = rest[NBUF:2 * NBUF]
        ssems = rest[2 * NBUF:]
        lead = LEAD
        wid = lax.axis_index("s") * 2 + lax.axis_index("c")
        base = wid * rows_per_worker
        pltpu.sync_copy(idx_hbm.at[pl.ds(base, rows_per_worker)], idx_v)

        for b in range(lead):
            pltpu.async_copy(
                table_hbm.at[idx_v.at[pl.ds(b * CHUNK, CHUNK)]], bufs[b], gsems[b]
            )

        def group(g, carry):
            goff = g * (NBUF * CHUNK)
            for b in range(NBUF):
                i_off = goff + b * CHUNK
                pltpu.make_async_copy(
                    table_hbm.at[idx_v.at[pl.ds(i_off, CHUNK)]], bufs[b], gsems[b]
                ).wait()
                pltpu.async_copy(
                    bufs[b], out_hbm.at[pl.ds(base + i_off, CHUNK)], ssems[b]
                )

                prev_off = i_off + (lead - NBUF) * CHUNK

                @pl.when(prev_off >= 0)
                def _():
                    pltpu.make_async_copy(
                        bufs[(b + lead) % NBUF],
                        out_hbm.at[pl.ds(base + prev_off, CHUNK)],
                        ssems[(b + lead) % NBUF],
                    ).wait()

                nxt_off = i_off + lead * CHUNK

                @pl.when(nxt_off < rows_per_worker)
                def _():
                    pltpu.async_copy(
                        table_hbm.at[idx_v.at[pl.ds(nxt_off, CHUNK)]],
                        bufs[(b + lead) % NBUF],
                        gsems[(b + lead) % NBUF],
                    )

            return carry

        lax.fori_loop(0, num_groups, group, 0)

        tail = NBUF - lead
        last = rows_per_worker - tail * CHUNK
        for b in range(tail):
            off = last + b * CHUNK
            pltpu.make_async_copy(
                bufs[(num_chunks - tail + b) % NBUF],
                out_hbm.at[pl.ds(base + off, CHUNK)],
                ssems[(num_chunks - tail + b) % NBUF],
            ).wait()

    return gather_kernel


import math

_NEG_LOG_TAU = -math.log(10000.0) / float(D_MODEL)
_ROWS_PER_TC_BLOCK = 512


_INV_2PI = 0.15915494309189535
_RED_C1 = 6.28125
_RED_C2 = 2.0 * math.pi - 6.28125
_SIN_COEFFS = (
    9.9999970703e-01,
    -1.6666577215e-01,
    8.3325581176e-03,
    -1.9812575520e-04,
    2.7040512124e-06,
    -2.0534244527e-08,
)


def _fast_sin(x):
    k = jnp.round(x * jnp.float32(_INV_2PI))
    y = (x - k * jnp.float32(_RED_C1)) - k * jnp.float32(_RED_C2)
    y2 = y * y
    poly = jnp.float32(_SIN_COEFFS[5])
    for c in _SIN_COEFFS[4::-1]:
        poly = poly * y2 + jnp.float32(c)
    return poly * y


def _formula_body(pos_ref, out_ref):
    p = pos_ref[...]
    col = lax.broadcasted_iota(jnp.int32, (1, D_MODEL), 1)
    w = jnp.exp((col & ~1).astype(jnp.float32) * _NEG_LOG_TAU)
    phase = (col & 1).astype(jnp.float32) * (math.pi / 2.0)
    x = (p - 1).astype(jnp.float32) * w + phase
    val = _fast_sin(x)
    out_ref[...] = jnp.where(p > 0, val, jnp.float32(0.0))


_SC_FRACTION_NUM = 1
_SC_FRACTION_DEN = 2


def _tc_formula(flat2d):
    n = flat2d.shape[0]
    srows = _ROWS_PER_TC_BLOCK
    return pl.pallas_call(
        _formula_body,
        grid=(n // srows,),
        in_specs=[pl.BlockSpec((srows, 1), lambda i: (i, 0))],
        out_specs=pl.BlockSpec((srows, D_MODEL), lambda i: (i, 0)),
        out_shape=jax.ShapeDtypeStruct((n, D_MODEL), jnp.float32),
    )(flat2d)


def kernel(positions, pe):
    b, s = positions.shape
    n = b * s
    flat = positions.reshape(n)
    out = _make_gather(n)(pe, flat)
    return out.reshape(b, s, pe.shape[1])


# speedup vs baseline: 1.0899x; 1.0899x over previous
import functools

import jax
import jax.numpy as jnp
from jax import lax
from jax.experimental import pallas as pl
from jax.experimental.pallas import tpu as pltpu
from jax.experimental.pallas import tpu_sc as plsc

D_MODEL = 1024
NUM_WORKERS = 32
CHUNK = 8
NBUF = 8
LEAD = 5


def _make_gather(batch):
    rows_per_worker = batch // NUM_WORKERS
    num_chunks = rows_per_worker // CHUNK
    num_groups = num_chunks // NBUF
    mesh = plsc.VectorSubcoreMesh(core_axis_name="c", subcore_axis_name="s")

    @functools.partial(
        pl.kernel,
        mesh=mesh,
        out_type=jax.ShapeDtypeStruct((batch, D_MODEL), jnp.float32),
        scratch_types=[
            pltpu.VMEM((rows_per_worker,), jnp.int32),
        ]
        + [pltpu.VMEM((CHUNK, D_MODEL), jnp.float32) for _ in range(NBUF)]
        + [pltpu.SemaphoreType.DMA for _ in range(2 * NBUF)],
    )
    def gather_kernel(table_hbm, idx_hbm, out_hbm, idx_v, *rest):
        bufs = rest[:NBUF]
        gsems = rest[NBUF:2 * NBUF]
        ssems = rest[2 * NBUF:]
        lead = LEAD
        wid = lax.axis_index("s") * 2 + lax.axis_index("c")
        base = wid * rows_per_worker
        pltpu.sync_copy(idx_hbm.at[pl.ds(base, rows_per_worker)], idx_v)

        for b in range(lead):
            pltpu.async_copy(
                table_hbm.at[idx_v.at[pl.ds(b * CHUNK, CHUNK)]], bufs[b], gsems[b]
            )

        def group(g, carry):
            goff = g * (NBUF * CHUNK)
            for b in range(NBUF):
                i_off = goff + b * CHUNK
                pltpu.make_async_copy(
                    table_hbm.at[idx_v.at[pl.ds(i_off, CHUNK)]], bufs[b], gsems[b]
                ).wait()
                pltpu.async_copy(
                    bufs[b], out_hbm.at[pl.ds(base + i_off, CHUNK)], ssems[b]
                )

                prev_off = i_off + (lead - NBUF) * CHUNK

                @pl.when(prev_off >= 0)
                def _():
                    pltpu.make_async_copy(
                        bufs[(b + lead) % NBUF],
                        out_hbm.at[pl.ds(base + prev_off, CHUNK)],
                        ssems[(b + lead) % NBUF],
                    ).wait()

                nxt_off = i_off + lead * CHUNK

                @pl.when(nxt_off < rows_per_worker)
                def _():
                    pltpu.async_copy(
                        table_hbm.at[idx_v.at[pl.ds(nxt_off, CHUNK)]],
                        bufs[(b + lead) % NBUF],
                        gsems[(b + lead) % NBUF],
                    )

            return carry

        lax.fori_loop(0, num_groups, group, 0)

        tail = NBUF - lead
        last = rows_per_worker - tail * CHUNK
        for b in range(tail):
            off = last + b * CHUNK
            pltpu.make_async_copy(
                bufs[(num_chunks - tail + b) % NBUF],
                out_hbm.at[pl.ds(base + off, CHUNK)],
                ssems[(num_chunks - tail + b) % NBUF],
            ).wait()

    return gather_kernel


import math

_NEG_LOG_TAU = -math.log(10000.0) / float(D_MODEL)
_ROWS_PER_TC_BLOCK = 512


_INV_2PI = 0.15915494309189535
_RED_C1 = 6.28125
_RED_C2 = 2.0 * math.pi - 6.28125
_SIN_COEFFS = (
    9.9999970703e-01,
    -1.6666577215e-01,
    8.3325581176e-03,
    -1.9812575520e-04,
    2.7040512124e-06,
    -2.0534244527e-08,
)


def _fast_sin(x):
    k = jnp.round(x * jnp.float32(_INV_2PI))
    y = (x - k * jnp.float32(_RED_C1)) - k * jnp.float32(_RED_C2)
    y2 = y * y
    poly = jnp.float32(_SIN_COEFFS[5])
    for c in _SIN_COEFFS[4::-1]:
        poly = poly * y2 + jnp.float32(c)
    return poly * y


def _formula_body(pos_ref, out_ref):
    p = pos_ref[...]
    col = lax.broadcasted_iota(jnp.int32, (1, D_MODEL), 1)
    w = jnp.exp((col & ~1).astype(jnp.float32) * _NEG_LOG_TAU)
    phase = (col & 1).astype(jnp.float32) * (math.pi / 2.0)
    x = (p - 1).astype(jnp.float32) * w + phase
    val = _fast_sin(x)
    out_ref[...] = jnp.where(p > 0, val, jnp.float32(0.0))


_SC_FRACTION_NUM = 1
_SC_FRACTION_DEN = 2


def _tc_formula(flat2d):
    n = flat2d.shape[0]
    srows = _ROWS_PER_TC_BLOCK
    return pl.pallas_call(
        _formula_body,
        grid=(n // srows,),
        in_specs=[pl.BlockSpec((srows, 1), lambda i: (i, 0))],
        out_specs=pl.BlockSpec((srows, D_MODEL), lambda i: (i, 0)),
        out_shape=jax.ShapeDtypeStruct((n, D_MODEL), jnp.float32),
    )(flat2d)


def kernel(positions, pe):
    b, s = positions.shape
    n = b * s
    flat = positions.reshape(n)
    out = _make_gather(n)(pe, flat)
    return out.reshape(b, s, pe.shape[1])
